# baseline (speedup 1.0000x reference)
import functools

import numpy as np
import jax
import jax.numpy as jnp
from jax import lax
from jax.experimental import pallas as pl
from jax.experimental.pallas import tpu as pltpu
from jax.experimental.pallas import tpu_sc as plsc

_NC = 2
_NS = 16
_NW = _NC * _NS
_P = 15
_G = 8
_F32 = jnp.float32


def _rup(n, m):
    return ((n + m - 1) // m) * m


def _mk_kpts(seed, radius, k=15):
    rng = np.random.RandomState(seed)
    pts = rng.randn(k, 3)
    pts = pts / (np.linalg.norm(pts, axis=1, keepdims=True) + 1e-9)
    r = rng.rand(k, 1) ** (1.0 / 3.0)
    return np.asarray(pts * r * radius, dtype=np.float32)


@functools.lru_cache(maxsize=None)
def _sc_gather_fn(S, D, B, chunk, n_chunks):
    b_per_w = n_chunks * chunk
    mesh = plsc.VectorSubcoreMesh(core_axis_name="c", subcore_axis_name="s")

    @functools.partial(
        pl.kernel,
        mesh=mesh,
        compiler_params=pltpu.CompilerParams(use_tc_tiling_on_sc=False),
        out_type=jax.ShapeDtypeStruct((B, D), jnp.float32),
        scratch_types=[
            pltpu.VMEM((chunk,), jnp.int32),
            pltpu.VMEM((chunk, D), jnp.float32),
            pltpu.SemaphoreType.DMA,
        ],
    )
    def k(table_hbm, idx_hbm, out_hbm, idx_v, rows_v, sem):
        wid = lax.axis_index("s") * _NC + lax.axis_index("c")
        base = wid * b_per_w

        def body(i, carry):
            off = base + i * chunk
            pltpu.sync_copy(idx_hbm.at[pl.ds(off, chunk)], idx_v)
            pltpu.async_copy(table_hbm.at[idx_v], rows_v, sem).wait()
            pltpu.sync_copy(rows_v, out_hbm.at[pl.ds(off, chunk)])
            return carry

        lax.fori_loop(0, n_chunks, body, 0)

    return k


def _sc_gather(table, idx_flat):
    S, D = table.shape
    B0 = idx_flat.shape[0]
    chunk = min(1024, max(8, (110000 // (D + 1)) // 8 * 8))
    per_rank = chunk * _NW
    n_chunks = -(-B0 // per_rank)
    B = n_chunks * per_rank
    if B != B0:
        idx_flat = jnp.pad(idx_flat, (0, B - B0))
    out = _sc_gather_fn(S, D, B, chunk, n_chunks)(table, idx_flat)
    return out[:B0]


def _gath(table, idx2d, n_rows_out):
    N, K = idx2d.shape
    D = table.shape[1]
    if n_rows_out != N:
        idx2d = jnp.pad(idx2d, ((0, n_rows_out - N), (0, 0)))
    flat = idx2d.reshape(-1)
    return _sc_gather(table, flat).reshape(n_rows_out, K, D)


def _pad16(x):
    D = x.shape[1]
    Dp = _rup(D, 16)
    if Dp != D:
        x = jnp.pad(x, ((0, 0), (0, Dp - D)))
    return x


def _gn(x, gamma, beta, C):
    cg = C // _G
    ci = lax.broadcasted_iota(jnp.int32, (C, _G), 0) // cg
    gi = lax.broadcasted_iota(jnp.int32, (C, _G), 1)
    M = (ci == gi).astype(_F32)
    s = jnp.dot(x, M, preferred_element_type=_F32) * (1.0 / cg)
    ss = jnp.dot(x * x, M, preferred_element_type=_F32) * (1.0 / cg)
    v = ss - s * s
    mean_c = jnp.dot(s, M.T, preferred_element_type=_F32)
    inv_c = jnp.dot(1.0 / jnp.sqrt(v + 1e-5), M.T, preferred_element_type=_F32)
    return (x - mean_c) * inv_c * gamma + beta


def _leaky(x):
    return jnp.where(x >= 0, x, 0.1 * x)


def _full_spec(shape):
    r = len(shape)
    return pl.BlockSpec(shape, lambda i, _r=r: (0,) * _r)


def _row_spec(T, shape):
    r = len(shape)
    return pl.BlockSpec((T,) + shape[1:], lambda i, _r=r: (i,) + (0,) * (_r - 1))


def _kpconv_tc(gp, gf, q, W, gamma, beta, kpts, sigma, fstart, C, T):
    Nt, K, Dp = gp.shape
    Cf = gf.shape[2]
    Dout = W.shape[2]
    kp = np.asarray(kpts)
    inv_sig = 1.0 / sigma

    def body(gp_ref, gf_ref, q_ref, kp_ref, W_ref, g_ref, b_ref, o_ref):
        gpv = gp_ref[...]
        qv = q_ref[...]
        sq = jnp.zeros((T, K, _P), _F32)
        for d in range(3):
            ndd = gpv[:, :, d:d + 1] - qv[:, :, d:d + 1]
            t = ndd - kp_ref[d:d + 1]
            sq = sq + t * t
        dist = jnp.sqrt(jnp.maximum(sq, 1e-12))
        infl = jnp.maximum(0.0, 1.0 - dist * inv_sig)
        f = gf_ref[...][:, :, fstart:fstart + C]
        acc = jnp.zeros((T, Dout), _F32)
        for p in range(_P):
            z = jnp.sum(infl[:, :, p:p + 1] * f, axis=1)
            acc = acc + jnp.dot(z, W_ref[p], preferred_element_type=_F32)
        o_ref[...] = _leaky(_gn(acc, g_ref[...], b_ref[...], Dout))

    return pl.pallas_call(
        body,
        grid=(Nt // T,),
        in_specs=[
            _row_spec(T, gp.shape),
            _row_spec(T, gf.shape),
            _row_spec(T, (q.shape[0], 1, 3)),
            _full_spec((3, 1, _P)),
            _full_spec(W.shape),
            _full_spec((1, Dout)),
            _full_spec((1, Dout)),
        ],
        out_specs=_row_spec(T, (Nt, Dout)),
        out_shape=jax.ShapeDtypeStruct((Nt, Dout), _F32),
    )(gp, gf, q.reshape(-1, 1, 3), jnp.asarray(kp.T.reshape(3, 1, _P)), W,
      gamma.reshape(1, -1), beta.reshape(1, -1))


def _unary_tc(x, W, b, gamma, beta, T, apply_leaky=True):
    Nt, Cin = x.shape
    Cout = W.shape[1]

    def body(x_ref, W_ref, b_ref, g_ref, be_ref, o_ref):
        y = jnp.dot(x_ref[...], W_ref[...], preferred_element_type=_F32) + b_ref[...]
        y = _gn(y, g_ref[...], be_ref[...], Cout)
        o_ref[...] = _leaky(y) if apply_leaky else y

    return pl.pallas_call(
        body,
        grid=(Nt // T,),
        in_specs=[
            _row_spec(T, x.shape),
            _full_spec(W.shape),
            _full_spec((1, Cout)),
            _full_spec((1, Cout)),
            _full_spec((1, Cout)),
        ],
        out_specs=_row_spec(T, (Nt, Cout)),
        out_shape=jax.ShapeDtypeStruct((Nt, Cout), _F32),
    )(x, W, b.reshape(1, -1), gamma.reshape(1, -1), beta.reshape(1, -1))


def _res_tail_tc(x2, sc, W2, b2, g2, be2, T, proj=None, pool=False):
    Nt = x2.shape[0]
    Cout = W2.shape[1]

    def body(*refs):
        if proj is not None:
            (x2_ref, sc_ref, W2_ref, b2_ref, g2_ref, be2_ref,
             sW_ref, sb_ref, sg_ref, sbe_ref, o_ref) = refs
        else:
            x2_ref, sc_ref, W2_ref, b2_ref, g2_ref, be2_ref, o_ref = refs
        y = jnp.dot(x2_ref[...], W2_ref[...], preferred_element_type=_F32) + b2_ref[...]
        y = _gn(y, g2_ref[...], be2_ref[...], Cout)
        if pool:
            s = jnp.max(sc_ref[...], axis=1)
        elif proj is not None:
            s = jnp.dot(sc_ref[...], sW_ref[...], preferred_element_type=_F32) + sb_ref[...]
            s = _gn(s, sg_ref[...], sbe_ref[...], Cout)
        else:
            s = sc_ref[...]
        o_ref[...] = _leaky(y + s)

    ins = [x2, sc, W2, b2.reshape(1, -1), g2.reshape(1, -1), be2.reshape(1, -1)]
    specs = [
        _row_spec(T, x2.shape),
        _row_spec(T, sc.shape),
        _full_spec(W2.shape),
        _full_spec((1, Cout)),
        _full_spec((1, Cout)),
        _full_spec((1, Cout)),
    ]
    if proj is not None:
        sW, sb, sg, sbe = proj
        ins += [sW, sb.reshape(1, -1), sg.reshape(1, -1), sbe.reshape(1, -1)]
        specs += [_full_spec(sW.shape), _full_spec((1, Cout)),
                  _full_spec((1, Cout)), _full_spec((1, Cout))]

    return pl.pallas_call(
        body,
        grid=(Nt // T,),
        in_specs=specs,
        out_specs=_row_spec(T, (Nt, Cout)),
        out_shape=jax.ShapeDtypeStruct((Nt, Cout), _F32),
    )(*ins)


def _decoder_tc(g, q, skip, Wi, Ws, b, gamma, beta, cs, T, head=None):
    Nt = g.shape[0]
    Cout = Wi.shape[1]

    def body(*refs):
        if head is not None:
            g_ref, q_ref, s_ref, Wi_ref, Ws_ref, b_ref, gm_ref, be_ref, oW_ref, ob_ref, o_ref = refs
        else:
            g_ref, q_ref, s_ref, Wi_ref, Ws_ref, b_ref, gm_ref, be_ref, o_ref = refs
        gv = g_ref[...]
        qv = q_ref[...]
        d2 = jnp.zeros((T, 3, 1), _F32)
        for d in range(3):
            t = gv[:, :, d:d + 1] - qv[:, :, d:d + 1]
            d2 = d2 + t * t
        w = 1.0 / (d2 + 1e-10)
        w = w / jnp.sum(w, axis=1, keepdims=True)
        interp = jnp.sum(w * gv[:, :, 3:3 + cs], axis=1)
        y = (jnp.dot(interp, Wi_ref[...], preferred_element_type=_F32)
             + jnp.dot(s_ref[...], Ws_ref[...], preferred_element_type=_F32)
             + b_ref[...])
        y = _leaky(_gn(y, gm_ref[...], be_ref[...], Cout))
        if head is not None:
            y = jnp.dot(y, oW_ref[...], preferred_element_type=_F32) + ob_ref[...]
        o_ref[...] = y

    Dfin = head[0].shape[1] if head is not None else Cout
    ins = [g, q.reshape(-1, 1, 3), skip, Wi, Ws, b.reshape(1, -1),
           gamma.reshape(1, -1), beta.reshape(1, -1)]
    specs = [
        _row_spec(T, g.shape),
        _row_spec(T, (q.shape[0], 1, 3)),
        _row_spec(T, skip.shape),
        _full_spec(Wi.shape),
        _full_spec(Ws.shape),
        _full_spec((1, Cout)),
        _full_spec((1, Cout)),
        _full_spec((1, Cout)),
    ]
    if head is not None:
        oW, ob = head
        ins += [oW, ob.reshape(1, -1)]
        specs += [_full_spec(oW.shape), _full_spec((1, Dfin))]

    return pl.pallas_call(
        body,
        grid=(Nt // T,),
        in_specs=specs,
        out_specs=_row_spec(T, (Nt, Dfin)),
        out_shape=jax.ShapeDtypeStruct((Nt, Dfin), _F32),
    )(*ins)


def kernel(feats, points0, points1, points2, points3,
           neighbors0, neighbors1, neighbors2, neighbors3,
           subsampling0, subsampling1, subsampling2,
           upsampling0, upsampling1, upsampling2, params):
    pts = [points0, points1, points2, points3]
    nbrs = [neighbors0, neighbors1, neighbors2, neighbors3]
    subs = [subsampling0, subsampling1, subsampling2]
    ups = [upsampling0, upsampling1, upsampling2]
    p = params

    N = [x.shape[0] for x in pts]
    Tn = [512, 512, 256, 128]
    Nt = [_rup(N[i], Tn[i]) for i in range(4)]

    r1 = 0.025 * 2.5
    s1 = 0.025 * 2.0
    sig = [s1 * (2 ** i) for i in range(4)]
    kp = [_mk_kpts(100 + i, r1 * (2 ** i)) for i in range(4)]

    qp = [jnp.pad(pts[i], ((0, Nt[i] - N[i]), (0, 0))) for i in range(4)]

    def res_block(qlvl, feats_in, gp, idx2d, rp, kpi, sgi, strided):
        mid = rp['u1W'].shape[1]
        T = Tn[qlvl]
        Tk = 128
        x1 = _unary_tc(feats_in, rp['u1W'], rp['u1b'], rp['u1g'], rp['u1be'],
                       Tn[qlvl] if not strided else Tn[qlvl - 1])
        gf = _gath(_pad16(x1), idx2d, Nt[qlvl])
        x2 = _kpconv_tc(gp, gf, qp[qlvl], rp['cW'], rp['cg'], rp['cbe'],
                        kpi, sgi, 0, mid, Tk)
        if strided:
            gpool = _gath(feats_in, idx2d, Nt[qlvl])
            return _res_tail_tc(x2, gpool, rp['u2W'], rp['u2b'], rp['u2g'],
                                rp['u2be'], Tk, pool=True)
        if 'sW' in rp:
            return _res_tail_tc(x2, feats_in, rp['u2W'], rp['u2b'], rp['u2g'],
                                rp['u2be'], T,
                                proj=(rp['sW'], rp['sb'], rp['sg'], rp['sbe']))
        return _res_tail_tc(x2, feats_in, rp['u2W'], rp['u2b'], rp['u2g'],
                            rp['u2be'], T)

    tbl0 = _pad16(jnp.concatenate([pts[0], feats], axis=1))
    g0 = _gath(tbl0, nbrs[0], Nt[0])
    f1a = _kpconv_tc(g0, g0, qp[0], p['e11']['cW'], p['e11']['cg'],
                     p['e11']['cbe'], kp[0], sig[0], 3, 1, 128)
    f1 = res_block(0, f1a, g0, nbrs[0], p['e12'], kp[0], sig[0], False)

    gs0 = _gath(_pad16(pts[0]), subs[0], Nt[1])
    f2 = res_block(1, f1, gs0, subs[0], p['e21'], kp[0], sig[0], True)
    g1 = _gath(_pad16(pts[1]), nbrs[1], Nt[1])
    f2 = res_block(1, f2, g1, nbrs[1], p['e22'], kp[1], sig[1], False)
    f2 = res_block(1, f2, g1, nbrs[1], p['e23'], kp[1], sig[1], False)

    gs1 = _gath(_pad16(pts[1]), subs[1], Nt[2])
    f3 = res_block(2, f2, gs1, subs[1], p['e31'], kp[1], sig[1], True)
    g2 = _gath(_pad16(pts[2]), nbrs[2], Nt[2])
    f3 = res_block(2, f3, g2, nbrs[2], p['e32'], kp[2], sig[2], False)
    f3 = res_block(2, f3, g2, nbrs[2], p['e33'], kp[2], sig[2], False)

    gs2 = _gath(_pad16(pts[2]), subs[2], Nt[3])
    f4 = res_block(3, f3, gs2, subs[2], p['e41'], kp[2], sig[2], True)
    g3 = _gath(_pad16(pts[3]), nbrs[3], Nt[3])
    f4 = res_block(3, f4, g3, nbrs[3], p['e42'], kp[3], sig[3], False)
    f4 = res_block(3, f4, g3, nbrs[3], p['e43'], kp[3], sig[3], False)

    l4 = f4

    def dec(qlvl, slvl, coarse_feats, skip, dp, head=None):
        cs = coarse_feats.shape[1]
        tab = _pad16(jnp.concatenate([pts[slvl], coarse_feats[:N[slvl]]], axis=1))
        gk = _gath(tab, ups[qlvl], Nt[qlvl])
        W = dp['W']
        return _decoder_tc(gk, qp[qlvl], skip, W[:cs], W[cs:], dp['b'],
                           dp['g'], dp['be'], cs, Tn[qlvl], head=head)

    l3 = dec(2, 3, l4, f3, p['d3'])
    l2 = dec(1, 2, l3, f2, p['d2'])
    l1f = dec(0, 1, l2, f1, p['d1'], head=(p['oW'], p['ob']))

    return (l1f[:N[0]], l2[:N[1]], l3[:N[2]], l4[:N[3]])


# speedup vs baseline: 1.2593x; 1.2593x over previous
import functools

import numpy as np
import jax
import jax.numpy as jnp
from jax import lax
from jax.experimental import pallas as pl
from jax.experimental.pallas import tpu as pltpu
from jax.experimental.pallas import tpu_sc as plsc

_NC = 2
_NS = 16
_NW = _NC * _NS
_P = 15
_G = 8
_F32 = jnp.float32


def _rup(n, m):
    return ((n + m - 1) // m) * m


def _mk_kpts(seed, radius, k=15):
    rng = np.random.RandomState(seed)
    pts = rng.randn(k, 3)
    pts = pts / (np.linalg.norm(pts, axis=1, keepdims=True) + 1e-9)
    r = rng.rand(k, 1) ** (1.0 / 3.0)
    return np.asarray(pts * r * radius, dtype=np.float32)


@functools.lru_cache(maxsize=None)
def _sc_gather_fn(S, D, B, chunk, n_chunks):
    b_per_w = n_chunks * chunk
    mesh = plsc.VectorSubcoreMesh(core_axis_name="c", subcore_axis_name="s")

    @functools.partial(
        pl.kernel,
        mesh=mesh,
        compiler_params=pltpu.CompilerParams(use_tc_tiling_on_sc=False),
        out_type=jax.ShapeDtypeStruct((B, D), jnp.float32),
        scratch_types=[
            pltpu.VMEM((chunk,), jnp.int32),
            pltpu.VMEM((chunk, D), jnp.float32),
            pltpu.SemaphoreType.DMA,
        ],
    )
    def k(table_hbm, idx_hbm, out_hbm, idx_v, rows_v, sem):
        wid = lax.axis_index("s") * _NC + lax.axis_index("c")
        base = wid * b_per_w

        def body(i, carry):
            off = base + i * chunk
            pltpu.sync_copy(idx_hbm.at[pl.ds(off, chunk)], idx_v)
            pltpu.async_copy(table_hbm.at[idx_v], rows_v, sem).wait()
            pltpu.sync_copy(rows_v, out_hbm.at[pl.ds(off, chunk)])
            return carry

        lax.fori_loop(0, n_chunks, body, 0)

    return k


def _sc_gather(table, idx_flat):
    S, D = table.shape
    B0 = idx_flat.shape[0]
    chunk = min(128, max(8, (110000 // (D + 1)) // 8 * 8))
    per_rank = chunk * _NW
    n_chunks = -(-B0 // per_rank)
    B = n_chunks * per_rank
    if B != B0:
        idx_flat = jnp.pad(idx_flat, (0, B - B0))
    out = _sc_gather_fn(S, D, B, chunk, n_chunks)(table, idx_flat)
    return out[:B0]


def _gath(table, idx2d, n_rows_out):
    N, K = idx2d.shape
    D = table.shape[1]
    if n_rows_out != N:
        idx2d = jnp.pad(idx2d, ((0, n_rows_out - N), (0, 0)))
    flat = idx2d.reshape(-1)
    return _sc_gather(table, flat).reshape(n_rows_out, K, D)


def _pad16(x):
    D = x.shape[1]
    Dp = _rup(D, 16)
    if Dp != D:
        x = jnp.pad(x, ((0, 0), (0, Dp - D)))
    return x


def _gn(x, gamma, beta, C):
    cg = C // _G
    ci = lax.broadcasted_iota(jnp.int32, (C, _G), 0) // cg
    gi = lax.broadcasted_iota(jnp.int32, (C, _G), 1)
    M = (ci == gi).astype(_F32)
    s = jnp.dot(x, M, preferred_element_type=_F32) * (1.0 / cg)
    ss = jnp.dot(x * x, M, preferred_element_type=_F32) * (1.0 / cg)
    v = ss - s * s
    mean_c = jnp.dot(s, M.T, preferred_element_type=_F32)
    inv_c = jnp.dot(1.0 / jnp.sqrt(v + 1e-5), M.T, preferred_element_type=_F32)
    return (x - mean_c) * inv_c * gamma + beta


def _leaky(x):
    return jnp.where(x >= 0, x, 0.1 * x)


def _full_spec(shape):
    r = len(shape)
    return pl.BlockSpec(shape, lambda i, _r=r: (0,) * _r)


def _row_spec(T, shape):
    r = len(shape)
    return pl.BlockSpec((T,) + shape[1:], lambda i, _r=r: (i,) + (0,) * (_r - 1))


def _kpconv_tc(gp, gf, q, W, gamma, beta, kpts, sigma, fstart, C, T):
    Nt, K, Dp = gp.shape
    Cf = gf.shape[2]
    Dout = W.shape[2]
    kp = np.asarray(kpts)
    inv_sig = 1.0 / sigma

    def body(gp_ref, gf_ref, q_ref, kp_ref, W_ref, g_ref, b_ref, o_ref):
        gpv = gp_ref[...]
        qv = q_ref[...]
        sq = jnp.zeros((T, K, _P), _F32)
        for d in range(3):
            ndd = gpv[:, :, d:d + 1] - qv[:, :, d:d + 1]
            t = ndd - kp_ref[d:d + 1]
            sq = sq + t * t
        dist = jnp.sqrt(jnp.maximum(sq, 1e-12))
        infl = jnp.maximum(0.0, 1.0 - dist * inv_sig)
        f = gf_ref[...][:, :, fstart:fstart + C]
        acc = jnp.zeros((T, Dout), _F32)
        for p in range(_P):
            z = jnp.sum(infl[:, :, p:p + 1] * f, axis=1)
            acc = acc + jnp.dot(z, W_ref[p], preferred_element_type=_F32)
        o_ref[...] = _leaky(_gn(acc, g_ref[...], b_ref[...], Dout))

    return pl.pallas_call(
        body,
        grid=(Nt // T,),
        in_specs=[
            _row_spec(T, gp.shape),
            _row_spec(T, gf.shape),
            _row_spec(T, (q.shape[0], 1, 3)),
            _full_spec((3, 1, _P)),
            _full_spec(W.shape),
            _full_spec((1, Dout)),
            _full_spec((1, Dout)),
        ],
        out_specs=_row_spec(T, (Nt, Dout)),
        out_shape=jax.ShapeDtypeStruct((Nt, Dout), _F32),
    )(gp, gf, q.reshape(-1, 1, 3), jnp.asarray(kp.T.reshape(3, 1, _P)), W,
      gamma.reshape(1, -1), beta.reshape(1, -1))


def _unary_tc(x, W, b, gamma, beta, T, apply_leaky=True):
    Nt, Cin = x.shape
    Cout = W.shape[1]

    def body(x_ref, W_ref, b_ref, g_ref, be_ref, o_ref):
        y = jnp.dot(x_ref[...], W_ref[...], preferred_element_type=_F32) + b_ref[...]
        y = _gn(y, g_ref[...], be_ref[...], Cout)
        o_ref[...] = _leaky(y) if apply_leaky else y

    return pl.pallas_call(
        body,
        grid=(Nt // T,),
        in_specs=[
            _row_spec(T, x.shape),
            _full_spec(W.shape),
            _full_spec((1, Cout)),
            _full_spec((1, Cout)),
            _full_spec((1, Cout)),
        ],
        out_specs=_row_spec(T, (Nt, Cout)),
        out_shape=jax.ShapeDtypeStruct((Nt, Cout), _F32),
    )(x, W, b.reshape(1, -1), gamma.reshape(1, -1), beta.reshape(1, -1))


def _res_tail_tc(x2, sc, W2, b2, g2, be2, T, proj=None, pool=False):
    Nt = x2.shape[0]
    Cout = W2.shape[1]

    def body(*refs):
        if proj is not None:
            (x2_ref, sc_ref, W2_ref, b2_ref, g2_ref, be2_ref,
             sW_ref, sb_ref, sg_ref, sbe_ref, o_ref) = refs
        else:
            x2_ref, sc_ref, W2_ref, b2_ref, g2_ref, be2_ref, o_ref = refs
        y = jnp.dot(x2_ref[...], W2_ref[...], preferred_element_type=_F32) + b2_ref[...]
        y = _gn(y, g2_ref[...], be2_ref[...], Cout)
        if pool:
            s = jnp.max(sc_ref[...], axis=1)
        elif proj is not None:
            s = jnp.dot(sc_ref[...], sW_ref[...], preferred_element_type=_F32) + sb_ref[...]
            s = _gn(s, sg_ref[...], sbe_ref[...], Cout)
        else:
            s = sc_ref[...]
        o_ref[...] = _leaky(y + s)

    ins = [x2, sc, W2, b2.reshape(1, -1), g2.reshape(1, -1), be2.reshape(1, -1)]
    specs = [
        _row_spec(T, x2.shape),
        _row_spec(T, sc.shape),
        _full_spec(W2.shape),
        _full_spec((1, Cout)),
        _full_spec((1, Cout)),
        _full_spec((1, Cout)),
    ]
    if proj is not None:
        sW, sb, sg, sbe = proj
        ins += [sW, sb.reshape(1, -1), sg.reshape(1, -1), sbe.reshape(1, -1)]
        specs += [_full_spec(sW.shape), _full_spec((1, Cout)),
                  _full_spec((1, Cout)), _full_spec((1, Cout))]

    return pl.pallas_call(
        body,
        grid=(Nt // T,),
        in_specs=specs,
        out_specs=_row_spec(T, (Nt, Cout)),
        out_shape=jax.ShapeDtypeStruct((Nt, Cout), _F32),
    )(*ins)


def _decoder_tc(g, q, skip, Wi, Ws, b, gamma, beta, cs, T, head=None):
    Nt = g.shape[0]
    Cout = Wi.shape[1]

    def body(*refs):
        if head is not None:
            g_ref, q_ref, s_ref, Wi_ref, Ws_ref, b_ref, gm_ref, be_ref, oW_ref, ob_ref, o_ref = refs
        else:
            g_ref, q_ref, s_ref, Wi_ref, Ws_ref, b_ref, gm_ref, be_ref, o_ref = refs
        gv = g_ref[...]
        qv = q_ref[...]
        d2 = jnp.zeros((T, 3, 1), _F32)
        for d in range(3):
            t = gv[:, :, d:d + 1] - qv[:, :, d:d + 1]
            d2 = d2 + t * t
        w = 1.0 / (d2 + 1e-10)
        w = w / jnp.sum(w, axis=1, keepdims=True)
        interp = jnp.sum(w * gv[:, :, 3:3 + cs], axis=1)
        y = (jnp.dot(interp, Wi_ref[...], preferred_element_type=_F32)
             + jnp.dot(s_ref[...], Ws_ref[...], preferred_element_type=_F32)
             + b_ref[...])
        y = _leaky(_gn(y, gm_ref[...], be_ref[...], Cout))
        if head is not None:
            y = jnp.dot(y, oW_ref[...], preferred_element_type=_F32) + ob_ref[...]
        o_ref[...] = y

    Dfin = head[0].shape[1] if head is not None else Cout
    ins = [g, q.reshape(-1, 1, 3), skip, Wi, Ws, b.reshape(1, -1),
           gamma.reshape(1, -1), beta.reshape(1, -1)]
    specs = [
        _row_spec(T, g.shape),
        _row_spec(T, (q.shape[0], 1, 3)),
        _row_spec(T, skip.shape),
        _full_spec(Wi.shape),
        _full_spec(Ws.shape),
        _full_spec((1, Cout)),
        _full_spec((1, Cout)),
        _full_spec((1, Cout)),
    ]
    if head is not None:
        oW, ob = head
        ins += [oW, ob.reshape(1, -1)]
        specs += [_full_spec(oW.shape), _full_spec((1, Dfin))]

    return pl.pallas_call(
        body,
        grid=(Nt // T,),
        in_specs=specs,
        out_specs=_row_spec(T, (Nt, Dfin)),
        out_shape=jax.ShapeDtypeStruct((Nt, Dfin), _F32),
    )(*ins)


def kernel(feats, points0, points1, points2, points3,
           neighbors0, neighbors1, neighbors2, neighbors3,
           subsampling0, subsampling1, subsampling2,
           upsampling0, upsampling1, upsampling2, params):
    pts = [points0, points1, points2, points3]
    nbrs = [neighbors0, neighbors1, neighbors2, neighbors3]
    subs = [subsampling0, subsampling1, subsampling2]
    ups = [upsampling0, upsampling1, upsampling2]
    p = params

    N = [x.shape[0] for x in pts]
    Tn = [512, 512, 256, 128]
    Nt = [_rup(N[i], Tn[i]) for i in range(4)]

    r1 = 0.025 * 2.5
    s1 = 0.025 * 2.0
    sig = [s1 * (2 ** i) for i in range(4)]
    kp = [_mk_kpts(100 + i, r1 * (2 ** i)) for i in range(4)]

    qp = [jnp.pad(pts[i], ((0, Nt[i] - N[i]), (0, 0))) for i in range(4)]

    def res_block(qlvl, feats_in, gp, idx2d, rp, kpi, sgi, strided):
        mid = rp['u1W'].shape[1]
        T = Tn[qlvl]
        Tk = 128
        x1 = _unary_tc(feats_in, rp['u1W'], rp['u1b'], rp['u1g'], rp['u1be'],
                       Tn[qlvl] if not strided else Tn[qlvl - 1])
        gf = _gath(_pad16(x1), idx2d, Nt[qlvl])
        x2 = _kpconv_tc(gp, gf, qp[qlvl], rp['cW'], rp['cg'], rp['cbe'],
                        kpi, sgi, 0, mid, Tk)
        if strided:
            gpool = _gath(feats_in, idx2d, Nt[qlvl])
            return _res_tail_tc(x2, gpool, rp['u2W'], rp['u2b'], rp['u2g'],
                                rp['u2be'], Tk, pool=True)
        if 'sW' in rp:
            return _res_tail_tc(x2, feats_in, rp['u2W'], rp['u2b'], rp['u2g'],
                                rp['u2be'], T,
                                proj=(rp['sW'], rp['sb'], rp['sg'], rp['sbe']))
        return _res_tail_tc(x2, feats_in, rp['u2W'], rp['u2b'], rp['u2g'],
                            rp['u2be'], T)

    tbl0 = _pad16(jnp.concatenate([pts[0], feats], axis=1))
    g0 = _gath(tbl0, nbrs[0], Nt[0])
    f1a = _kpconv_tc(g0, g0, qp[0], p['e11']['cW'], p['e11']['cg'],
                     p['e11']['cbe'], kp[0], sig[0], 3, 1, 128)
    f1 = res_block(0, f1a, g0, nbrs[0], p['e12'], kp[0], sig[0], False)

    gs0 = _gath(_pad16(pts[0]), subs[0], Nt[1])
    f2 = res_block(1, f1, gs0, subs[0], p['e21'], kp[0], sig[0], True)
    g1 = _gath(_pad16(pts[1]), nbrs[1], Nt[1])
    f2 = res_block(1, f2, g1, nbrs[1], p['e22'], kp[1], sig[1], False)
    f2 = res_block(1, f2, g1, nbrs[1], p['e23'], kp[1], sig[1], False)

    gs1 = _gath(_pad16(pts[1]), subs[1], Nt[2])
    f3 = res_block(2, f2, gs1, subs[1], p['e31'], kp[1], sig[1], True)
    g2 = _gath(_pad16(pts[2]), nbrs[2], Nt[2])
    f3 = res_block(2, f3, g2, nbrs[2], p['e32'], kp[2], sig[2], False)
    f3 = res_block(2, f3, g2, nbrs[2], p['e33'], kp[2], sig[2], False)

    gs2 = _gath(_pad16(pts[2]), subs[2], Nt[3])
    f4 = res_block(3, f3, gs2, subs[2], p['e41'], kp[2], sig[2], True)
    g3 = _gath(_pad16(pts[3]), nbrs[3], Nt[3])
    f4 = res_block(3, f4, g3, nbrs[3], p['e42'], kp[3], sig[3], False)
    f4 = res_block(3, f4, g3, nbrs[3], p['e43'], kp[3], sig[3], False)

    l4 = f4

    def dec(qlvl, slvl, coarse_feats, skip, dp, head=None):
        cs = coarse_feats.shape[1]
        tab = _pad16(jnp.concatenate([pts[slvl], coarse_feats[:N[slvl]]], axis=1))
        gk = _gath(tab, ups[qlvl], Nt[qlvl])
        W = dp['W']
        return _decoder_tc(gk, qp[qlvl], skip, W[:cs], W[cs:], dp['b'],
                           dp['g'], dp['be'], cs, Tn[qlvl], head=head)

    l3 = dec(2, 3, l4, f3, p['d3'])
    l2 = dec(1, 2, l3, f2, p['d2'])
    l1f = dec(0, 1, l2, f1, p['d1'], head=(p['oW'], p['ob']))

    return (l1f[:N[0]], l2[:N[1]], l3[:N[2]], l4[:N[3]])
